# Initial kernel scaffold; baseline (speedup 1.0000x reference)
import jax
import jax.numpy as jnp
from jax.experimental import pallas as pl


def kernel(z, emb):
    raise NotImplementedError("write your pallas kernel here")


# speedup vs baseline: 1.5113x; 1.5113x over previous
import functools

import jax
import jax.numpy as jnp
from jax import lax
from jax.experimental import pallas as pl
from jax.experimental.pallas import tpu as pltpu
from jax.experimental.pallas import tpu_sc as plsc

_NUM_CODES = 8192
_DIM = 256
_BETA = 0.25
_ROW_BLOCK = 256


def _tc_body(z_ref, emb_ref, idx_ref, d2sum_ref, bb_ref):
    i = pl.program_id(0)

    @pl.when(i == 0)
    def _():
        e = emb_ref[...]
        bb_ref[...] = jnp.sum(e * e, axis=1)[None, :]
        d2sum_ref[0, 0] = 0.0

    z = z_ref[...]
    ab = lax.dot_general(
        z, emb_ref[...], (((1,), (1,)), ((), ())),
        preferred_element_type=jnp.float32,
    )
    aa = jnp.sum(z * z, axis=1, keepdims=True)
    d2 = aa + bb_ref[...] - 2.0 * ab
    m = jnp.min(d2, axis=1, keepdims=True)
    mc = jnp.maximum(m, 0.0)
    s = jnp.sqrt(mc)
    b0i = lax.bitcast_convert_type(s * s, jnp.int32)
    thr = m
    for k in range(0, 5):
        bk = lax.bitcast_convert_type(b0i + k, jnp.float32)
        ok = jnp.sqrt(jnp.maximum(bk, 0.0)) == s
        thr = jnp.maximum(thr, jnp.where(ok, bk, -jnp.inf))
    iota = lax.broadcasted_iota(jnp.int32, d2.shape, 1).astype(jnp.float32)
    cand = jnp.where(d2 <= thr, iota, jnp.float32(2.0 ** 24))
    idx_ref[...] = jnp.min(cand, axis=1).astype(jnp.int32)
    d2sum_ref[0, 0] += jnp.sum(mc[:, 0])


@jax.jit
def _tc_argmin(z_flat, emb):
    n_rows = z_flat.shape[0]
    grid = (n_rows // _ROW_BLOCK,)
    return pl.pallas_call(
        _tc_body,
        grid=grid,
        in_specs=[
            pl.BlockSpec((_ROW_BLOCK, _DIM), lambda i: (i, 0)),
            pl.BlockSpec((_NUM_CODES, _DIM), lambda i: (0, 0)),
        ],
        out_specs=[
            pl.BlockSpec((_ROW_BLOCK,), lambda i: (i,)),
            pl.BlockSpec((1, 1), lambda i: (0, 0), memory_space=pltpu.SMEM),
        ],
        out_shape=[
            jax.ShapeDtypeStruct((n_rows,), jnp.int32),
            jax.ShapeDtypeStruct((1, 1), jnp.float32),
        ],
        scratch_shapes=[pltpu.VMEM((1, _NUM_CODES), jnp.float32)],
    )(z_flat, emb)


def _make_sc_gather(n_rows):
    info = plsc.get_sparse_core_info()
    nw = info.num_cores * info.num_subcores
    b_per_w = n_rows // nw
    mesh = plsc.VectorSubcoreMesh(core_axis_name="c", subcore_axis_name="s")
    chunk = 96
    assert b_per_w % chunk == 0
    n_chunks = b_per_w // chunk

    @functools.partial(
        pl.kernel,
        mesh=mesh,
        out_type=jax.ShapeDtypeStruct((n_rows, _DIM), jnp.float32),
        scratch_types=[
            pltpu.VMEM((b_per_w,), jnp.int32),
            pltpu.VMEM((b_per_w, _DIM), jnp.float32),
            pltpu.SemaphoreType.DMA,
        ],
    )
    def gather(idx_hbm, table_hbm, out_hbm, idx_v, rows_v, sem):
        wid = lax.axis_index("s") * info.num_cores + lax.axis_index("c")
        base = wid * b_per_w
        pltpu.sync_copy(idx_hbm.at[pl.ds(base, b_per_w)], idx_v)
        copies = [
            pltpu.async_copy(
                table_hbm.at[idx_v.at[pl.ds(c * chunk, chunk)]],
                rows_v.at[pl.ds(c * chunk, chunk)],
                sem,
            )
            for c in range(n_chunks)
        ]
        for cp in copies:
            cp.wait()
        pltpu.sync_copy(rows_v, out_hbm.at[pl.ds(base, b_per_w)])

    return gather


_sc_gather_cache = {}


def _sc_gather(indices, emb):
    n_rows = indices.shape[0]
    if n_rows not in _sc_gather_cache:
        _sc_gather_cache[n_rows] = _make_sc_gather(n_rows)
    return _sc_gather_cache[n_rows](indices, emb)


def kernel(z, emb):
    bz, n, dim = z.shape
    z_flat = z.reshape(-1, dim)
    indices, d2sum = _tc_argmin(z_flat, emb)
    z_q = _sc_gather(indices, emb).reshape(z.shape)
    loss = d2sum[0, 0] * ((1.0 + _BETA) / z.size)
    return (z_q, indices, loss)
